# baseline (speedup 1.0000x reference)
import jax
import jax.numpy as jnp
from jax.experimental import pallas as pl
import jax.experimental.pallas.tpu as pltpu

N_HEADS = 16
MAX_REL = 32
SEQ_LEN = 2048
EXT = 2 * SEQ_LEN
ROWS = 2048
TCOL = 256
SHEAR_W = TCOL + ROWS


def _bias_kernel(delta_ref, table_t_ref, out_ref, e_ref):
    h = pl.program_id(0)
    ib = pl.program_id(1)
    delta = delta_ref[0]

    @pl.when(ib == 0)
    def _build_e():
        m = jax.lax.broadcasted_iota(jnp.int32, (64, EXT), 1)
        r = jax.lax.broadcasted_iota(jnp.int32, (64, EXT), 0)
        idx = jnp.clip(MAX_REL - 1 + SEQ_LEN - 1 - m + delta, 0, 2 * MAX_REL - 2)
        one_hot = (idx == r).astype(jnp.float32)
        e_ref[...] = jnp.dot(table_t_ref[...], one_hot,
                             preferred_element_type=jnp.float32)

    i0 = ib * ROWS
    e_row = e_ref[pl.ds(h, 1), :]
    e_row = pltpu.roll(e_row, (EXT - (SEQ_LEN - 1) + i0) % EXT, 1)
    block = jnp.broadcast_to(e_row, (ROWS, EXT))
    rolled = pltpu.roll(block, 0, 1, stride=1, stride_axis=0)
    out_ref[0, 0, :, :] = rolled[:, :SEQ_LEN]


def kernel(seq_len_q, seq_len_k, relative_position_bias):
    delta = (jnp.asarray(seq_len_q, jnp.int32) - jnp.asarray(seq_len_k, jnp.int32)
             ).reshape((1,))
    table_t = jnp.zeros((N_HEADS, 64), jnp.float32).at[:, : 2 * MAX_REL - 1].set(
        relative_position_bias.T
    )

    grid = (N_HEADS, SEQ_LEN // ROWS)
    out = pl.pallas_call(
        _bias_kernel,
        grid=grid,
        in_specs=[pl.BlockSpec(memory_space=pltpu.SMEM),
                  pl.BlockSpec((N_HEADS, 64), lambda h, ib: (0, 0))],
        out_specs=pl.BlockSpec((1, 1, ROWS, SEQ_LEN), lambda h, ib: (0, h, ib, 0)),
        out_shape=jax.ShapeDtypeStruct((1, N_HEADS, SEQ_LEN, SEQ_LEN), jnp.float32),
        scratch_shapes=[pltpu.VMEM((N_HEADS, EXT), jnp.float32)],
        compiler_params=pltpu.CompilerParams(
            dimension_semantics=("parallel", "arbitrary")),
    )(delta, table_t)
    return out


# speedup vs baseline: 1.0145x; 1.0145x over previous
import jax
import jax.numpy as jnp
from jax.experimental import pallas as pl
import jax.experimental.pallas.tpu as pltpu

N_HEADS = 16
MAX_REL = 32
SEQ_LEN = 2048
EXT = 2 * SEQ_LEN
ROWS = 1024


def _bias_kernel(delta_ref, table_t_ref, out_ref, e_ref):
    h = pl.program_id(0)
    ib = pl.program_id(1)
    delta = delta_ref[0]

    @pl.when(ib == 0)
    def _build_e():
        m = jax.lax.broadcasted_iota(jnp.int32, (64, EXT), 1)
        r = jax.lax.broadcasted_iota(jnp.int32, (64, EXT), 0)
        idx = jnp.clip(MAX_REL - 1 + SEQ_LEN - 1 - m + delta, 0, 2 * MAX_REL - 2)
        one_hot = (idx == r).astype(jnp.float32)
        e_ref[...] = jnp.dot(table_t_ref[...], one_hot,
                             preferred_element_type=jnp.float32)

    i0 = ib * ROWS
    e_row = e_ref[pl.ds(h, 1), :]
    e_row = pltpu.roll(e_row, (EXT - (SEQ_LEN - 1) + i0) % EXT, 1)
    block = jnp.broadcast_to(e_row, (ROWS, EXT))
    rolled = pltpu.roll(block, 0, 1, stride=1, stride_axis=0)
    out_ref[0, 0, :, :] = rolled[:, :SEQ_LEN]


def kernel(seq_len_q, seq_len_k, relative_position_bias):
    delta = (jnp.asarray(seq_len_q, jnp.int32) - jnp.asarray(seq_len_k, jnp.int32)
             ).reshape((1,))
    table_t = jnp.zeros((N_HEADS, 64), jnp.float32).at[:, : 2 * MAX_REL - 1].set(
        relative_position_bias.T
    )

    grid = (N_HEADS, SEQ_LEN // ROWS)
    out = pl.pallas_call(
        _bias_kernel,
        grid=grid,
        in_specs=[pl.BlockSpec(memory_space=pltpu.SMEM),
                  pl.BlockSpec((N_HEADS, 64), lambda h, ib: (0, 0))],
        out_specs=pl.BlockSpec((1, 1, ROWS, SEQ_LEN), lambda h, ib: (0, h, ib, 0)),
        out_shape=jax.ShapeDtypeStruct((1, N_HEADS, SEQ_LEN, SEQ_LEN), jnp.float32),
        scratch_shapes=[pltpu.VMEM((N_HEADS, EXT), jnp.float32)],
        compiler_params=pltpu.CompilerParams(
            dimension_semantics=("parallel", "arbitrary")),
    )(delta, table_t)
    return out
